# baseline (speedup 1.0000x reference)
import functools

import jax
import jax.numpy as jnp
from jax.experimental import pallas as pl
from jax.experimental.pallas import tpu as pltpu
from jax.experimental.pallas import tpu_sc as plsc

_W = 512
_SPLITS = 2


def _gather_call(table2, idx, n, emb):
    mesh = plsc.VectorSubcoreMesh(core_axis_name="c", subcore_axis_name="s")

    @functools.partial(
        pl.kernel,
        out_type=jax.ShapeDtypeStruct((n, emb), jnp.float32),
        mesh=mesh,
        compiler_params=pltpu.CompilerParams(use_tc_tiling_on_sc=False),
    )
    def gather_kernel(table_hbm, idx_hbm, out_hbm):
        def body(i_vmem, o_vmem):
            pltpu.sync_copy(table_hbm.at[i_vmem.at[0]], o_vmem)

        pltpu.emit_pipeline(
            body,
            grid=(n // _W,),
            in_specs=[pl.BlockSpec((1, _W), index_map=lambda i: (0, i))],
            out_specs=[pl.BlockSpec((_W, emb), index_map=lambda i: (i, 0))],
            core_axis_name=("c", "s"),
            dimension_semantics=(pltpu.PARALLEL,),
        )(idx_hbm, out_hbm)

    return gather_kernel(table2.reshape(table2.shape[0] * 2, emb), idx)


def kernel(inputs, table):
    b, l = inputs.shape
    v, emb = table.shape
    table2 = table.reshape(v // 2, emb * 2)

    lc = l // _SPLITS
    outs = []
    for s in range(_SPLITS):
        idx_s = inputs[:, s * lc:(s + 1) * lc].reshape(1, b * lc)
        out_s = _gather_call(table2, idx_s, b * lc, emb)
        outs.append(out_s.reshape(b, lc, emb))
    return jnp.concatenate(outs, axis=1)


# speedup vs baseline: 6.5161x; 6.5161x over previous
import functools

import jax
import jax.numpy as jnp
from jax.experimental import pallas as pl
from jax.experimental.pallas import tpu as pltpu
from jax.experimental.pallas import tpu_sc as plsc

_W = 512


def kernel(inputs, table):
    b, l = inputs.shape
    v, emb = table.shape
    n = b * l
    idx = inputs.reshape(1, n)
    table_pairs = jax.lax.optimization_barrier(table.reshape(v // 2, emb * 2))
    table_lin = table_pairs.reshape(v, emb)

    mesh = plsc.VectorSubcoreMesh(core_axis_name="c", subcore_axis_name="s")

    @functools.partial(
        pl.kernel,
        out_type=jax.ShapeDtypeStruct((n, emb), table.dtype),
        mesh=mesh,
        compiler_params=pltpu.CompilerParams(use_tc_tiling_on_sc=False),
    )
    def gather_kernel(table_hbm, idx_hbm, out_hbm):
        def body(i_vmem, o_vmem):
            pltpu.sync_copy(table_hbm.at[i_vmem.at[0]], o_vmem)

        pltpu.emit_pipeline(
            body,
            grid=(n // _W,),
            in_specs=[pl.BlockSpec((1, _W), index_map=lambda i: (0, i))],
            out_specs=[pl.BlockSpec((_W, emb), index_map=lambda i: (i, 0))],
            core_axis_name=("c", "s"),
            dimension_semantics=(pltpu.PARALLEL,),
        )(idx_hbm, out_hbm)

    out = gather_kernel(table_lin, idx)
    return out.reshape(b, l, emb)


# speedup vs baseline: 6.9498x; 1.0666x over previous
import functools

import jax
import jax.numpy as jnp
from jax.experimental import pallas as pl
from jax.experimental.pallas import tpu as pltpu
from jax.experimental.pallas import tpu_sc as plsc

_W = 512


def kernel(inputs, table):
    b, l = inputs.shape
    v, emb = table.shape
    n = b * l
    idx = inputs.reshape(1, n)
    t2 = (
        table.T.reshape(emb, v // 2, 2)
        .transpose(1, 2, 0)
        .reshape(v // 2, emb * 2)
    )
    table_lin = t2.reshape(v, emb)

    mesh = plsc.VectorSubcoreMesh(core_axis_name="c", subcore_axis_name="s")

    @functools.partial(
        pl.kernel,
        out_type=jax.ShapeDtypeStruct((n, emb), table.dtype),
        mesh=mesh,
        compiler_params=pltpu.CompilerParams(use_tc_tiling_on_sc=False),
    )
    def gather_kernel(table_hbm, idx_hbm, out_hbm):
        def body(i_vmem, o_vmem):
            pltpu.sync_copy(table_hbm.at[i_vmem.at[0]], o_vmem)

        pltpu.emit_pipeline(
            body,
            grid=(n // _W,),
            in_specs=[pl.BlockSpec((1, _W), index_map=lambda i: (0, i))],
            out_specs=[pl.BlockSpec((_W, emb), index_map=lambda i: (i, 0))],
            core_axis_name=("c", "s"),
            dimension_semantics=(pltpu.PARALLEL,),
        )(idx_hbm, out_hbm)

    out = gather_kernel(table_lin, idx)
    return out.reshape(b, l, emb)
